# baseline (speedup 1.0000x reference)
import functools

import jax
import jax.numpy as jnp
from jax import lax
from jax.experimental import pallas as pl
from jax.experimental.pallas import tpu as pltpu
from jax.experimental.pallas import tpu_sc as plsc

B = 16384
U_DIM = 64
M_DIM = 64
NUM_USERS = 1000000
NUM_MOVIES = 100000
BU_ROWS = (NUM_USERS + 127) // 128
BM_ROWS = (NUM_MOVIES + 127) // 128

_NC, _NS = 2, 16
NW = _NC * _NS
BPW = B // NW
CH = 128
NCH = BPW // CH


def _sc_gather_body(uid2_h, mid2_h, uid7_h, mid7_h,
                    eu_h, em_h, bu_h, bm_h,
                    out_u, out_m, out_bu, out_bm,
                    uidx2, midx2, uidx7, midx7,
                    urows, mrows, bugr, bmgr, sem):
    wid = lax.axis_index("s") * _NC + lax.axis_index("c")
    base = wid * BPW
    pltpu.sync_copy(uid2_h.at[pl.ds(base, BPW)], uidx2)
    pltpu.sync_copy(mid2_h.at[pl.ds(base, BPW)], midx2)
    pltpu.sync_copy(uid7_h.at[pl.ds(base, BPW)], uidx7)
    pltpu.sync_copy(mid7_h.at[pl.ds(base, BPW)], midx7)
    for c in range(NCH):
        o = c * CH
        c1 = pltpu.async_copy(eu_h.at[uidx2.at[pl.ds(o, CH)]], urows, sem)
        c2 = pltpu.async_copy(em_h.at[midx2.at[pl.ds(o, CH)]], mrows, sem)
        c3 = pltpu.async_copy(bu_h.at[uidx7.at[pl.ds(o, CH)]], bugr, sem)
        c4 = pltpu.async_copy(bm_h.at[midx7.at[pl.ds(o, CH)]], bmgr, sem)
        c1.wait()
        c2.wait()
        c3.wait()
        c4.wait()
        pltpu.sync_copy(urows, out_u.at[pl.ds(base + o, CH)])
        pltpu.sync_copy(mrows, out_m.at[pl.ds(base + o, CH)])
        pltpu.sync_copy(bugr, out_bu.at[pl.ds(base + o, CH)])
        pltpu.sync_copy(bmgr, out_bm.at[pl.ds(base + o, CH)])


@functools.cache
def _sc_gather():
    return pl.kernel(
        _sc_gather_body,
        out_type=[
            jax.ShapeDtypeStruct((B, 128), jnp.float32),
            jax.ShapeDtypeStruct((B, 128), jnp.float32),
            jax.ShapeDtypeStruct((B, 128), jnp.float32),
            jax.ShapeDtypeStruct((B, 128), jnp.float32),
        ],
        mesh=plsc.VectorSubcoreMesh(core_axis_name="c", subcore_axis_name="s",
                                    num_cores=_NC, num_subcores=_NS),
        scratch_types=[
            pltpu.VMEM((BPW,), jnp.int32),
            pltpu.VMEM((BPW,), jnp.int32),
            pltpu.VMEM((BPW,), jnp.int32),
            pltpu.VMEM((BPW,), jnp.int32),
            pltpu.VMEM((CH, 128), jnp.float32),
            pltpu.VMEM((CH, 128), jnp.float32),
            pltpu.VMEM((CH, 128), jnp.float32),
            pltpu.VMEM((CH, 128), jnp.float32),
            pltpu.SemaphoreType.DMA,
        ],
    )


BB = 2048


def _tc_dense_body(uid_ref, mid_ref, age_ref, gen_ref, catsT_ref,
                   u128_ref, m128_ref, bu128_ref, bm128_ref,
                   ea_ref, eg_ref, ec_ref, uw_ref, ub_ref, mw_ref, mb_ref,
                   out_ref):
    f32 = jnp.float32
    uw = uw_ref[...]
    mw = mw_ref[...]
    a_age = jnp.dot(ea_ref[...], uw[64:80, :], preferred_element_type=f32)
    a_gen = jnp.dot(eg_ref[...], uw[80:96, :], preferred_element_type=f32)
    a_cat = jnp.dot(ec_ref[...], mw[64:96, :], preferred_element_type=f32)

    upar = (uid_ref[...] & 1)[:, None] == 1
    mpar = (mid_ref[...] & 1)[:, None] == 1
    ru = jnp.where(upar, u128_ref[:, 64:128], u128_ref[:, 0:64])
    rm = jnp.where(mpar, m128_ref[:, 64:128], m128_ref[:, 0:64])

    age = age_ref[...]
    gen = gen_ref[...]
    aoh = (age[:, None] == lax.broadcasted_iota(jnp.int32, (BB, 8), 1)).astype(f32)
    goh = (gen[:, None] == lax.broadcasted_iota(jnp.int32, (BB, 4), 1)).astype(f32)

    uv = (jnp.dot(ru, uw[:64, :], preferred_element_type=f32)
          + jnp.dot(aoh, a_age, preferred_element_type=f32)
          + jnp.dot(goh, a_gen, preferred_element_type=f32)
          + ub_ref[...][None, :])

    iota32 = lax.broadcasted_iota(jnp.int32, (BB, 32), 1)
    coh = jnp.zeros((BB, 32), dtype=f32)
    for j in range(8):
        coh = coh + (catsT_ref[j, :][:, None] == iota32).astype(f32)
    coh = coh * (iota32 != 0).astype(f32)
    cnt = jnp.maximum(jnp.sum(coh, axis=1, keepdims=True), 1.0)
    coh = coh / cnt

    mv = (jnp.dot(rm, mw[:64, :], preferred_element_type=f32)
          + jnp.dot(coh, a_cat, preferred_element_type=f32)
          + mb_ref[...][None, :])

    iota128 = lax.broadcasted_iota(jnp.int32, (BB, 128), 1)
    uoh = ((uid_ref[...] & 127)[:, None] == iota128).astype(f32)
    moh = ((mid_ref[...] & 127)[:, None] == iota128).astype(f32)
    bu = jnp.sum(bu128_ref[...] * uoh, axis=1)
    bm = jnp.sum(bm128_ref[...] * moh, axis=1)

    s = jnp.sum(uv * mv, axis=1) + bu + bm
    p = 1.0 / (1.0 + jnp.exp(-s))
    out_ref[...] = p * (1.0 + 2 * 0.1) - 0.1


def _tc_dense(uid, mid, user_age, user_gender, catsT, u128, m128,
              bu128, bm128, emb_age, emb_gender, emb_movie_cats,
              user_fc_w, user_fc_b, movie_fc_w, movie_fc_b):
    grid = (B // BB,)
    full = lambda i: (0, 0)
    return pl.pallas_call(
        _tc_dense_body,
        grid=grid,
        in_specs=[
            pl.BlockSpec((BB,), lambda i: (i,)),
            pl.BlockSpec((BB,), lambda i: (i,)),
            pl.BlockSpec((BB,), lambda i: (i,)),
            pl.BlockSpec((BB,), lambda i: (i,)),
            pl.BlockSpec((8, BB), lambda i: (0, i)),
            pl.BlockSpec((BB, 128), lambda i: (i, 0)),
            pl.BlockSpec((BB, 128), lambda i: (i, 0)),
            pl.BlockSpec((BB, 128), lambda i: (i, 0)),
            pl.BlockSpec((BB, 128), lambda i: (i, 0)),
            pl.BlockSpec((8, 16), full),
            pl.BlockSpec((4, 16), full),
            pl.BlockSpec((32, 32), full),
            pl.BlockSpec((96, 20), full),
            pl.BlockSpec((20,), lambda i: (0,)),
            pl.BlockSpec((96, 20), full),
            pl.BlockSpec((20,), lambda i: (0,)),
        ],
        out_specs=pl.BlockSpec((BB,), lambda i: (i,)),
        out_shape=jax.ShapeDtypeStruct((B,), jnp.float32),
    )(uid, mid, user_age, user_gender, catsT, u128, m128, bu128, bm128,
      emb_age, emb_gender, emb_movie_cats,
      user_fc_w, user_fc_b, movie_fc_w, movie_fc_b)


def kernel(user_id, user_age, user_gender, movie_id, movie_categories,
           emb_users, emb_movies, emb_movie_cats, emb_age, emb_gender,
           bias_user, bias_movie, user_fc_w, user_fc_b, movie_fc_w, movie_fc_b):
    uid = user_id.astype(jnp.int32)
    mid = movie_id.astype(jnp.int32)
    eu128 = jnp.concatenate([emb_users[0::2], emb_users[1::2]], axis=1)
    em128 = jnp.concatenate([emb_movies[0::2], emb_movies[1::2]], axis=1)
    bu_pad = jnp.pad(bias_user.reshape(-1), (0, BU_ROWS * 128 - NUM_USERS))
    bm_pad = jnp.pad(bias_movie.reshape(-1), (0, BM_ROWS * 128 - NUM_MOVIES))
    u128, m128, bu128, bm128 = _sc_gather()(
        uid >> 1, mid >> 1, uid >> 7, mid >> 7,
        eu128, em128, bu_pad.reshape(BU_ROWS, 128), bm_pad.reshape(BM_ROWS, 128))
    catsT = movie_categories.astype(jnp.int32).T
    return _tc_dense(uid, mid,
                     user_age.astype(jnp.int32), user_gender.astype(jnp.int32),
                     catsT, u128, m128, bu128, bm128,
                     emb_age, emb_gender, emb_movie_cats,
                     user_fc_w, user_fc_b, movie_fc_w, movie_fc_b)


# speedup vs baseline: 13.2892x; 13.2892x over previous
import functools

import jax
import jax.numpy as jnp
from jax import lax
from jax.experimental import pallas as pl
from jax.experimental.pallas import tpu as pltpu
from jax.experimental.pallas import tpu_sc as plsc

B = 16384
U_DIM = 64
M_DIM = 64
NUM_USERS = 1000000
NUM_MOVIES = 100000
BU_ROWS = NUM_USERS // 16
BM_ROWS = NUM_MOVIES // 16

_NC, _NS = 2, 16
NW = _NC * _NS
BPW = B // NW
CH = 128
NCH = BPW // CH


def _sc_gather_body(uid_h, mid_h, uid4_h, mid4_h,
                    eu_h, em_h, bu_h, bm_h,
                    out_u, out_m, out_bu, out_bm,
                    uidx, midx, uidx4, midx4,
                    urows, mrows, bugr, bmgr, sem):
    wid = lax.axis_index("s") * _NC + lax.axis_index("c")
    base = wid * BPW
    pltpu.sync_copy(uid_h.at[pl.ds(base, BPW)], uidx)
    pltpu.sync_copy(mid_h.at[pl.ds(base, BPW)], midx)
    pltpu.sync_copy(uid4_h.at[pl.ds(base, BPW)], uidx4)
    pltpu.sync_copy(mid4_h.at[pl.ds(base, BPW)], midx4)
    for c in range(NCH):
        o = c * CH
        c1 = pltpu.async_copy(eu_h.at[uidx.at[pl.ds(o, CH)]], urows, sem)
        c2 = pltpu.async_copy(em_h.at[midx.at[pl.ds(o, CH)]], mrows, sem)
        c3 = pltpu.async_copy(bu_h.at[uidx4.at[pl.ds(o, CH)]], bugr, sem)
        c4 = pltpu.async_copy(bm_h.at[midx4.at[pl.ds(o, CH)]], bmgr, sem)
        c1.wait()
        c2.wait()
        c3.wait()
        c4.wait()
        pltpu.sync_copy(urows, out_u.at[pl.ds(base + o, CH)])
        pltpu.sync_copy(mrows, out_m.at[pl.ds(base + o, CH)])
        pltpu.sync_copy(bugr, out_bu.at[pl.ds(base + o, CH)])
        pltpu.sync_copy(bmgr, out_bm.at[pl.ds(base + o, CH)])


@functools.cache
def _sc_gather():
    return pl.kernel(
        _sc_gather_body,
        out_type=[
            jax.ShapeDtypeStruct((B, U_DIM), jnp.float32),
            jax.ShapeDtypeStruct((B, M_DIM), jnp.float32),
            jax.ShapeDtypeStruct((B, 16), jnp.float32),
            jax.ShapeDtypeStruct((B, 16), jnp.float32),
        ],
        mesh=plsc.VectorSubcoreMesh(core_axis_name="c", subcore_axis_name="s",
                                    num_cores=_NC, num_subcores=_NS),
        scratch_types=[
            pltpu.VMEM((BPW,), jnp.int32),
            pltpu.VMEM((BPW,), jnp.int32),
            pltpu.VMEM((BPW,), jnp.int32),
            pltpu.VMEM((BPW,), jnp.int32),
            pltpu.VMEM((CH, U_DIM), jnp.float32),
            pltpu.VMEM((CH, M_DIM), jnp.float32),
            pltpu.VMEM((CH, 16), jnp.float32),
            pltpu.VMEM((CH, 16), jnp.float32),
            pltpu.SemaphoreType.DMA,
        ],
        compiler_params=pltpu.CompilerParams(use_tc_tiling_on_sc=False),
    )


BB = 2048


def _tc_dense_body(uid_ref, mid_ref, age_ref, gen_ref, catsT_ref,
                   ru_ref, rm_ref, bu16_ref, bm16_ref,
                   ea_ref, eg_ref, ec_ref, uw_ref, ub_ref, mw_ref, mb_ref,
                   out_ref):
    f32 = jnp.float32
    uw = uw_ref[...]
    mw = mw_ref[...]
    a_age = jnp.dot(ea_ref[...], uw[64:80, :], preferred_element_type=f32)
    a_gen = jnp.dot(eg_ref[...], uw[80:96, :], preferred_element_type=f32)
    a_cat = jnp.dot(ec_ref[...], mw[64:96, :], preferred_element_type=f32)

    age = age_ref[...]
    gen = gen_ref[...]
    aoh = (age[:, None] == lax.broadcasted_iota(jnp.int32, (BB, 8), 1)).astype(f32)
    goh = (gen[:, None] == lax.broadcasted_iota(jnp.int32, (BB, 4), 1)).astype(f32)

    uv = (jnp.dot(ru_ref[...], uw[:64, :], preferred_element_type=f32)
          + jnp.dot(aoh, a_age, preferred_element_type=f32)
          + jnp.dot(goh, a_gen, preferred_element_type=f32)
          + ub_ref[...][None, :])

    iota32 = lax.broadcasted_iota(jnp.int32, (BB, 32), 1)
    coh = jnp.zeros((BB, 32), dtype=f32)
    for j in range(8):
        coh = coh + (catsT_ref[j, :][:, None] == iota32).astype(f32)
    coh = coh * (iota32 != 0).astype(f32)
    cnt = jnp.maximum(jnp.sum(coh, axis=1, keepdims=True), 1.0)
    coh = coh / cnt

    mv = (jnp.dot(rm_ref[...], mw[:64, :], preferred_element_type=f32)
          + jnp.dot(coh, a_cat, preferred_element_type=f32)
          + mb_ref[...][None, :])

    iota16 = lax.broadcasted_iota(jnp.int32, (BB, 16), 1)
    uoh = ((uid_ref[...] & 15)[:, None] == iota16).astype(f32)
    moh = ((mid_ref[...] & 15)[:, None] == iota16).astype(f32)
    bu = jnp.sum(bu16_ref[...] * uoh, axis=1)
    bm = jnp.sum(bm16_ref[...] * moh, axis=1)

    s = jnp.sum(uv * mv, axis=1) + bu + bm
    p = 1.0 / (1.0 + jnp.exp(-s))
    out_ref[...] = p * (1.0 + 2 * 0.1) - 0.1


def _tc_dense(uid, mid, user_age, user_gender, catsT, ru, rm,
              bu16, bm16, emb_age, emb_gender, emb_movie_cats,
              user_fc_w, user_fc_b, movie_fc_w, movie_fc_b):
    grid = (B // BB,)
    full = lambda i: (0, 0)
    return pl.pallas_call(
        _tc_dense_body,
        grid=grid,
        in_specs=[
            pl.BlockSpec((BB,), lambda i: (i,)),
            pl.BlockSpec((BB,), lambda i: (i,)),
            pl.BlockSpec((BB,), lambda i: (i,)),
            pl.BlockSpec((BB,), lambda i: (i,)),
            pl.BlockSpec((8, BB), lambda i: (0, i)),
            pl.BlockSpec((BB, U_DIM), lambda i: (i, 0)),
            pl.BlockSpec((BB, M_DIM), lambda i: (i, 0)),
            pl.BlockSpec((BB, 16), lambda i: (i, 0)),
            pl.BlockSpec((BB, 16), lambda i: (i, 0)),
            pl.BlockSpec((8, 16), full),
            pl.BlockSpec((4, 16), full),
            pl.BlockSpec((32, 32), full),
            pl.BlockSpec((96, 20), full),
            pl.BlockSpec((20,), lambda i: (0,)),
            pl.BlockSpec((96, 20), full),
            pl.BlockSpec((20,), lambda i: (0,)),
        ],
        out_specs=pl.BlockSpec((BB,), lambda i: (i,)),
        out_shape=jax.ShapeDtypeStruct((B,), jnp.float32),
    )(uid, mid, user_age, user_gender, catsT, ru, rm, bu16, bm16,
      emb_age, emb_gender, emb_movie_cats,
      user_fc_w, user_fc_b, movie_fc_w, movie_fc_b)


def kernel(user_id, user_age, user_gender, movie_id, movie_categories,
           emb_users, emb_movies, emb_movie_cats, emb_age, emb_gender,
           bias_user, bias_movie, user_fc_w, user_fc_b, movie_fc_w, movie_fc_b):
    uid = user_id.astype(jnp.int32)
    mid = movie_id.astype(jnp.int32)
    ru, rm, bu16, bm16 = _sc_gather()(
        uid, mid, uid >> 4, mid >> 4,
        emb_users, emb_movies,
        bias_user.reshape(BU_ROWS, 16), bias_movie.reshape(BM_ROWS, 16))
    catsT = movie_categories.astype(jnp.int32).T
    return _tc_dense(uid, mid,
                     user_age.astype(jnp.int32), user_gender.astype(jnp.int32),
                     catsT, ru, rm, bu16, bm16,
                     emb_age, emb_gender, emb_movie_cats,
                     user_fc_w, user_fc_b, movie_fc_w, movie_fc_b)


# speedup vs baseline: 14.1463x; 1.0645x over previous
import functools

import jax
import jax.numpy as jnp
from jax import lax
from jax.experimental import pallas as pl
from jax.experimental.pallas import tpu as pltpu
from jax.experimental.pallas import tpu_sc as plsc

B = 16384
U_DIM = 64
M_DIM = 64
NUM_USERS = 1000000
NUM_MOVIES = 100000

_NC, _NS = 2, 16
NW = _NC * _NS
BPW = B // NW
CH = 128
NCH = BPW // CH


def _sc_gather_body(uid_h, mid_h, eu_h, em_h, bu_h, bm_h,
                    out_u, out_m, out_bu, out_bm,
                    uidx, midx, urows, mrows, buv, bmv, sem):
    wid = lax.axis_index("s") * _NC + lax.axis_index("c")
    base = wid * BPW
    pltpu.sync_copy(uid_h.at[pl.ds(base, BPW)], uidx)
    pltpu.sync_copy(mid_h.at[pl.ds(base, BPW)], midx)
    for c in range(NCH):
        o = c * CH

        def enq(g, _, o=o):
            uvec = uidx[pl.ds(o + g * 16, 16)]
            mvec = midx[pl.ds(o + g * 16, 16)]
            for k in range(16):
                ui = uvec[k]
                mi = mvec[k]
                s = g * 16 + k
                pltpu.async_copy(eu_h.at[pl.ds(ui, 1)],
                                 urows.at[pl.ds(s, 1)], sem)
                pltpu.async_copy(em_h.at[pl.ds(mi, 1)],
                                 mrows.at[pl.ds(s, 1)], sem)
                pltpu.async_copy(bu_h.at[pl.ds(ui, 1)],
                                 buv.at[pl.ds(s, 1)], sem)
                pltpu.async_copy(bm_h.at[pl.ds(mi, 1)],
                                 bmv.at[pl.ds(s, 1)], sem)
            return 0

        def drain(g, _, o=o):
            uvec = uidx[pl.ds(o + g * 16, 16)]
            mvec = midx[pl.ds(o + g * 16, 16)]
            for k in range(16):
                ui = uvec[k]
                mi = mvec[k]
                s = g * 16 + k
                pltpu.make_async_copy(eu_h.at[pl.ds(ui, 1)],
                                      urows.at[pl.ds(s, 1)], sem).wait()
                pltpu.make_async_copy(em_h.at[pl.ds(mi, 1)],
                                      mrows.at[pl.ds(s, 1)], sem).wait()
                pltpu.make_async_copy(bu_h.at[pl.ds(ui, 1)],
                                      buv.at[pl.ds(s, 1)], sem).wait()
                pltpu.make_async_copy(bm_h.at[pl.ds(mi, 1)],
                                      bmv.at[pl.ds(s, 1)], sem).wait()
            return 0

        lax.fori_loop(0, CH // 16, enq, 0)
        lax.fori_loop(0, CH // 16, drain, 0)
        pltpu.sync_copy(urows, out_u.at[pl.ds(base + o, CH)])
        pltpu.sync_copy(mrows, out_m.at[pl.ds(base + o, CH)])
        pltpu.sync_copy(buv, out_bu.at[pl.ds(base + o, CH)])
        pltpu.sync_copy(bmv, out_bm.at[pl.ds(base + o, CH)])


@functools.cache
def _sc_gather():
    return pl.kernel(
        _sc_gather_body,
        out_type=[
            jax.ShapeDtypeStruct((B, U_DIM), jnp.float32),
            jax.ShapeDtypeStruct((B, M_DIM), jnp.float32),
            jax.ShapeDtypeStruct((B, 1), jnp.float32),
            jax.ShapeDtypeStruct((B, 1), jnp.float32),
        ],
        mesh=plsc.VectorSubcoreMesh(core_axis_name="c", subcore_axis_name="s",
                                    num_cores=_NC, num_subcores=_NS),
        scratch_types=[
            pltpu.VMEM((BPW,), jnp.int32),
            pltpu.VMEM((BPW,), jnp.int32),
            pltpu.VMEM((CH, U_DIM), jnp.float32),
            pltpu.VMEM((CH, M_DIM), jnp.float32),
            pltpu.VMEM((CH, 1), jnp.float32),
            pltpu.VMEM((CH, 1), jnp.float32),
            pltpu.SemaphoreType.DMA,
        ],
    )


BB = 2048


def _tc_dense_body(age_ref, gen_ref, catsT_ref,
                   ru_ref, rm_ref, bu_ref, bm_ref,
                   ea_ref, eg_ref, ec_ref, uw_ref, ub_ref, mw_ref, mb_ref,
                   out_ref):
    f32 = jnp.float32
    uw = uw_ref[...]
    mw = mw_ref[...]
    a_age = jnp.dot(ea_ref[...], uw[64:80, :], preferred_element_type=f32)
    a_gen = jnp.dot(eg_ref[...], uw[80:96, :], preferred_element_type=f32)
    a_cat = jnp.dot(ec_ref[...], mw[64:96, :], preferred_element_type=f32)

    age = age_ref[...]
    gen = gen_ref[...]
    aoh = (age[:, None] == lax.broadcasted_iota(jnp.int32, (BB, 8), 1)).astype(f32)
    goh = (gen[:, None] == lax.broadcasted_iota(jnp.int32, (BB, 4), 1)).astype(f32)

    uv = (jnp.dot(ru_ref[...], uw[:64, :], preferred_element_type=f32)
          + jnp.dot(aoh, a_age, preferred_element_type=f32)
          + jnp.dot(goh, a_gen, preferred_element_type=f32)
          + ub_ref[...][None, :])

    iota32 = lax.broadcasted_iota(jnp.int32, (BB, 32), 1)
    coh = jnp.zeros((BB, 32), dtype=f32)
    for j in range(8):
        coh = coh + (catsT_ref[j, :][:, None] == iota32).astype(f32)
    coh = coh * (iota32 != 0).astype(f32)
    cnt = jnp.maximum(jnp.sum(coh, axis=1, keepdims=True), 1.0)
    coh = coh / cnt

    mv = (jnp.dot(rm_ref[...], mw[:64, :], preferred_element_type=f32)
          + jnp.dot(coh, a_cat, preferred_element_type=f32)
          + mb_ref[...][None, :])

    s = jnp.sum(uv * mv, axis=1) + bu_ref[...][:, 0] + bm_ref[...][:, 0]
    p = 1.0 / (1.0 + jnp.exp(-s))
    out_ref[...] = p * (1.0 + 2 * 0.1) - 0.1


def _tc_dense(user_age, user_gender, catsT, ru, rm,
              bu, bm, emb_age, emb_gender, emb_movie_cats,
              user_fc_w, user_fc_b, movie_fc_w, movie_fc_b):
    grid = (B // BB,)
    full = lambda i: (0, 0)
    return pl.pallas_call(
        _tc_dense_body,
        grid=grid,
        in_specs=[
            pl.BlockSpec((BB,), lambda i: (i,)),
            pl.BlockSpec((BB,), lambda i: (i,)),
            pl.BlockSpec((8, BB), lambda i: (0, i)),
            pl.BlockSpec((BB, U_DIM), lambda i: (i, 0)),
            pl.BlockSpec((BB, M_DIM), lambda i: (i, 0)),
            pl.BlockSpec((BB, 1), lambda i: (i, 0)),
            pl.BlockSpec((BB, 1), lambda i: (i, 0)),
            pl.BlockSpec((8, 16), full),
            pl.BlockSpec((4, 16), full),
            pl.BlockSpec((32, 32), full),
            pl.BlockSpec((96, 20), full),
            pl.BlockSpec((20,), lambda i: (0,)),
            pl.BlockSpec((96, 20), full),
            pl.BlockSpec((20,), lambda i: (0,)),
        ],
        out_specs=pl.BlockSpec((BB,), lambda i: (i,)),
        out_shape=jax.ShapeDtypeStruct((B,), jnp.float32),
    )(user_age, user_gender, catsT, ru, rm, bu, bm,
      emb_age, emb_gender, emb_movie_cats,
      user_fc_w, user_fc_b, movie_fc_w, movie_fc_b)


def kernel(user_id, user_age, user_gender, movie_id, movie_categories,
           emb_users, emb_movies, emb_movie_cats, emb_age, emb_gender,
           bias_user, bias_movie, user_fc_w, user_fc_b, movie_fc_w, movie_fc_b):
    uid = user_id.astype(jnp.int32)
    mid = movie_id.astype(jnp.int32)
    ru, rm, bu, bm = _sc_gather()(
        uid, mid, emb_users, emb_movies, bias_user, bias_movie)
    catsT = movie_categories.astype(jnp.int32).T
    return _tc_dense(user_age.astype(jnp.int32), user_gender.astype(jnp.int32),
                     catsT, ru, rm, bu, bm,
                     emb_age, emb_gender, emb_movie_cats,
                     user_fc_w, user_fc_b, movie_fc_w, movie_fc_b)
